# baseline (speedup 1.0000x reference)
import jax
import jax.numpy as jnp
from jax import lax
from jax.experimental import pallas as pl
from jax.experimental.pallas import tpu as pltpu
from jax.experimental.pallas import tpu_sc as plsc

B = 8
N = 4096
C = 128
NG = 1024
K = 32
R2 = float(0.2 ** 2)
NC = 2
NSUB = 16
QN = N // 4
NCHUNK = N // 16
SG = NG // 4
BIGI = 2 ** 31 - 1


def _iota16():
    return lax.broadcasted_iota(jnp.int32, (16,), 0)


def _round_bf16(v):
    u = plsc.bitcast(v, jnp.int32)
    r = (u + 0x7FFF + ((u >> 16) & 1)) & jnp.int32(-65536)
    return plsc.bitcast(r, jnp.float32)


def _sc_body(xt_hbm, f_hbm, fce_hbm, pce_hbm,
             xv, yv, zv, xbv, ybv, zbv, nqv, distv, centf, pub, comb, nbr,
             gidxa, gidxb, rowsa, rowsb, outf, shared, sema, semb):
    c = lax.axis_index("c")
    s = lax.axis_index("s")
    b = c * 4 + s // 4
    q = s % 4
    sgrp = (s // 4) * 4
    iota = _iota16()

    pltpu.sync_copy(xt_hbm.at[pl.ds((b * 3 + 0) * N, N)], xv.at[pl.ds(0, N)])
    pltpu.sync_copy(xt_hbm.at[pl.ds((b * 3 + 1) * N, N)], yv.at[pl.ds(0, N)])
    pltpu.sync_copy(xt_hbm.at[pl.ds((b * 3 + 2) * N, N)], zv.at[pl.ds(0, N)])

    def _init_chunk(j, _):
        xq = xv[pl.ds(j * 16, 16)]
        yq = yv[pl.ds(j * 16, 16)]
        zq = zv[pl.ds(j * 16, 16)]
        nqv[pl.ds(j * 16, 16)] = (xq * xq + yq * yq) + zq * zq
        xbv[pl.ds(j * 16, 16)] = _round_bf16(xq)
        ybv[pl.ds(j * 16, 16)] = _round_bf16(yq)
        zbv[pl.ds(j * 16, 16)] = _round_bf16(zq)
        return 0

    lax.fori_loop(0, NCHUNK, _init_chunk, 0)

    def _init_dist(j, _):
        distv[pl.ds(j * 16, 16)] = jnp.full((16,), 1e10, jnp.float32)
        return 0

    lax.fori_loop(0, QN // 16, _init_dist, 0)

    qoff = q * QN
    cmask = iota < 3

    def fps_step(i, carry):
        fx, fy, fz = carry
        vcent = jnp.where(iota == 0, fx, jnp.where(iota == 1, fy, fz))
        plsc.store_scatter(centf, [i * 3 + iota], vcent, mask=cmask)

        @plsc.parallel_loop(
            0, QN // 16, 1, unroll=4,
            carry=(jnp.full((16,), -1.0, jnp.float32),
                   jnp.full((16,), BIGI, jnp.int32)))
        def upd(j, bcarry):
            bv, bi = bcarry
            goff = qoff + j * 16
            xq = xv[pl.ds(goff, 16)]
            yq = yv[pl.ds(goff, 16)]
            zq = zv[pl.ds(goff, 16)]
            dx = xq - fx
            dy = yq - fy
            dz = zq - fz
            d2 = (dx * dx + dy * dy) + dz * dz
            dm = jnp.minimum(distv[pl.ds(j * 16, 16)], d2)
            distv[pl.ds(j * 16, 16)] = dm
            gix = goff + iota
            take = (dm > bv) | ((dm == bv) & (gix < bi))
            bv = jnp.where(take, dm, bv)
            bi = jnp.where(take, gix, bi)
            return bv, bi

        bv, bi = upd

        maxv = lax.reduce_max(bv, axes=(0,))
        cand = jnp.where(bv == maxv, bi, jnp.int32(BIGI))
        mini = lax.reduce_min(cand, axes=(0,))
        wvx = xv[pl.ds(mini, 16)]
        wvy = yv[pl.ds(mini, 16)]
        wvz = zv[pl.ds(mini, 16)]
        wx = wvx[0]
        wy = wvy[0]
        wz = wvz[0]
        pubv = jnp.where(
            iota == 0, maxv,
            jnp.where(iota == 1, mini.astype(jnp.float32),
                      jnp.where(iota == 2, wx,
                                jnp.where(iota == 3, wy, wz))))
        pub[...] = pubv
        par = i % 2
        pltpu.sync_copy(pub, shared.at[par, pl.ds(s * 16, 16)])
        plsc.subcore_barrier()
        pltpu.sync_copy(shared.at[par, pl.ds(sgrp * 16, 64)], comb)

        bvs = jnp.float32(-1.0)
        bis = jnp.float32(2.0 ** 30)
        nx = fx
        ny = fy
        nz = fz
        for w in range(4):
            row = comb[pl.ds(w * 16, 16)]
            v = row[0]
            ixf = row[1]
            better = (v > bvs) | ((v == bvs) & (ixf < bis))
            bvs = jnp.where(better, v, bvs)
            bis = jnp.where(better, ixf, bis)
            nx = jnp.where(better, row[2], nx)
            ny = jnp.where(better, row[3], ny)
            nz = jnp.where(better, row[4], nz)
        return nx, ny, nz

    v0x = xv[pl.ds(0, 16)]
    v0y = yv[pl.ds(0, 16)]
    v0z = zv[pl.ds(0, 16)]
    lax.fori_loop(0, NG, fps_step, (v0x[0], v0y[0], v0z[0]))

    boff = b * N
    r2 = jnp.float32(R2)

    def ballquery(cid, gd):
        cv = centf[pl.ds(cid * 3, 16)]
        cvb = _round_bf16(cv)
        cx = cv[0]
        cy = cv[1]
        cz = cv[2]
        ss = (cx * cx + cy * cy) + cz * cz
        cbx = cvb[0]
        cby = cvb[1]
        cbz = cvb[2]

        def chunk_mask(j):
            xq = xbv[pl.ds(j * 16, 16)]
            yq = ybv[pl.ds(j * 16, 16)]
            zq = zbv[pl.ds(j * 16, 16)]
            nq = nqv[pl.ds(j * 16, 16)]
            dot = (cbx * xq + cby * yq) + cbz * zq
            d2 = (-2.0 * dot + ss) + nq
            mask = d2 <= r2
            pc = plsc.all_reduce_population_count(mask)
            return mask, pc[0]

        def append(j, cnt, mask, pcs):
            @pl.when(pcs > 0)
            def _():
                cs = plsc.cumsum(mask.astype(jnp.int32))
                plsc.store_scatter(nbr, [(cnt - 1) + cs], j * 16 + iota,
                                   mask=mask)

        def cond(st):
            j, cnt = st
            return (cnt < K) & (j < NCHUNK)

        def scan(st):
            j, cnt = st
            m0, p0 = chunk_mask(j)
            m1, p1 = chunk_mask(j + 1)
            append(j, cnt, m0, p0)
            append(j + 1, cnt + p0, m1, p1)
            return j + 2, cnt + p0 + p1

        _, cnt = lax.while_loop(cond, scan, (jnp.int32(0), jnp.int32(0)))

        v1 = nbr[pl.ds(0, 16)]
        v2 = nbr[pl.ds(16, 16)]
        first = v1[0]
        gd[pl.ds(0, 16)] = jnp.where(iota < cnt, v1, first) + boff
        gd[pl.ds(16, 16)] = jnp.where(iota + 16 < cnt, v2, first) + boff

    def reduce_store(rws, si):
        accs = [rws[0, pl.ds(g * 16, 16)] for g in range(8)]
        for r in range(1, K):
            accs = [jnp.maximum(accs[g], rws[r, pl.ds(g * 16, 16)])
                    for g in range(8)]
        obase = si * C
        for g in range(8):
            outf[pl.ds(obase + g * 16, 16)] = accs[g]

    ballquery(q * SG, gidxa)
    pltpu.async_copy(f_hbm.at[gidxa], rowsa, sema)

    def pipe(si2, _):
        s0 = 2 * si2
        ballquery(q * SG + s0 + 1, gidxb)
        pltpu.async_copy(f_hbm.at[gidxb], rowsb, semb)
        pltpu.make_async_copy(f_hbm.at[gidxa], rowsa, sema).wait()
        reduce_store(rowsa, s0)

        @pl.when(s0 + 2 < SG)
        def _():
            ballquery(q * SG + s0 + 2, gidxa)
            pltpu.async_copy(f_hbm.at[gidxa], rowsa, sema)

        pltpu.make_async_copy(f_hbm.at[gidxb], rowsb, semb).wait()
        reduce_store(rowsb, s0 + 1)
        return 0

    lax.fori_loop(0, SG // 2, pipe, 0)

    pltpu.sync_copy(outf, fce_hbm.at[pl.ds((b * NG + q * SG) * C, SG * C)])

    @pl.when(q == 0)
    def _():
        pltpu.sync_copy(centf.at[pl.ds(0, NG * 3)],
                        pce_hbm.at[pl.ds(b * NG * 3, NG * 3)])


@jax.jit
def _transition_down(f, p):
    xt = jnp.transpose(p, (0, 2, 1)).reshape(B * 3 * N)
    f2 = f.reshape(B * N, C)
    mesh = plsc.VectorSubcoreMesh(
        core_axis_name="c", subcore_axis_name="s",
        num_cores=NC, num_subcores=NSUB)
    fce, pce = pl.kernel(
        _sc_body,
        out_type=(
            jax.ShapeDtypeStruct((B * NG * C,), jnp.float32),
            jax.ShapeDtypeStruct((B * NG * 3,), jnp.float32),
        ),
        mesh=mesh,
        compiler_params=pltpu.CompilerParams(needs_layout_passes=False),
        scratch_types=[
            pltpu.VMEM((N + 16,), jnp.float32),
            pltpu.VMEM((N + 16,), jnp.float32),
            pltpu.VMEM((N + 16,), jnp.float32),
            pltpu.VMEM((N,), jnp.float32),
            pltpu.VMEM((N,), jnp.float32),
            pltpu.VMEM((N,), jnp.float32),
            pltpu.VMEM((N,), jnp.float32),
            pltpu.VMEM((QN,), jnp.float32),
            pltpu.VMEM((NG * 3 + 16,), jnp.float32),
            pltpu.VMEM((16,), jnp.float32),
            pltpu.VMEM((64,), jnp.float32),
            pltpu.VMEM((64,), jnp.int32),
            pltpu.VMEM((K,), jnp.int32),
            pltpu.VMEM((K,), jnp.int32),
            pltpu.VMEM((K, C), jnp.float32),
            pltpu.VMEM((K, C), jnp.float32),
            pltpu.VMEM((SG * C,), jnp.float32),
            pltpu.VMEM_SHARED((2, NSUB * 16), jnp.float32),
            pltpu.SemaphoreType.DMA,
            pltpu.SemaphoreType.DMA,
        ],
    )(xt, f2)
    return fce.reshape(B, NG, C), pce.reshape(B, NG, 3)


def kernel(f, p):
    if f.shape[1] == NG:
        return (f, p)
    return _transition_down(f, p)


# speedup vs baseline: 1.2227x; 1.2227x over previous
import jax
import jax.numpy as jnp
from jax import lax
from jax.experimental import pallas as pl
from jax.experimental.pallas import tpu as pltpu
from jax.experimental.pallas import tpu_sc as plsc

B = 8
N = 4096
C = 128
NG = 1024
K = 32
R2 = float(0.2 ** 2)
NC = 2
NSUB = 16
QN = N // 4
NCHUNK = N // 16
SG = NG // 4
BIGI = 2 ** 31 - 1


def _iota16():
    return lax.broadcasted_iota(jnp.int32, (16,), 0)


def _round_bf16(v):
    u = plsc.bitcast(v, jnp.int32)
    r = (u + 0x7FFF + ((u >> 16) & 1)) & jnp.int32(-65536)
    return plsc.bitcast(r, jnp.float32)


def _sc_body(xt_hbm, f_hbm, fce_hbm, pce_hbm,
             xv, yv, zv, xbv, ybv, zbv, nqv, distv, centf, pub, comb, nbr,
             gidxa, gidxb, rowsa, rowsb, outf, shared, sema, semb):
    c = lax.axis_index("c")
    s = lax.axis_index("s")
    b = c * 4 + s // 4
    q = s % 4
    sgrp = (s // 4) * 4
    iota = _iota16()

    pltpu.sync_copy(xt_hbm.at[pl.ds((b * 3 + 0) * N, N)], xv.at[pl.ds(0, N)])
    pltpu.sync_copy(xt_hbm.at[pl.ds((b * 3 + 1) * N, N)], yv.at[pl.ds(0, N)])
    pltpu.sync_copy(xt_hbm.at[pl.ds((b * 3 + 2) * N, N)], zv.at[pl.ds(0, N)])

    def _init_chunk(j, _):
        xq = xv[pl.ds(j * 16, 16)]
        yq = yv[pl.ds(j * 16, 16)]
        zq = zv[pl.ds(j * 16, 16)]
        nqv[pl.ds(j * 16, 16)] = (xq * xq + yq * yq) + zq * zq
        xbv[pl.ds(j * 16, 16)] = _round_bf16(xq)
        ybv[pl.ds(j * 16, 16)] = _round_bf16(yq)
        zbv[pl.ds(j * 16, 16)] = _round_bf16(zq)
        return 0

    lax.fori_loop(0, NCHUNK, _init_chunk, 0)

    def _init_dist(j, _):
        distv[pl.ds(j * 16, 16)] = jnp.full((16,), 1e10, jnp.float32)
        return 0

    lax.fori_loop(0, QN // 16, _init_dist, 0)

    qoff = q * QN
    cmask = iota < 3

    def fps_step(i, carry):
        fx, fy, fz = carry
        vcent = jnp.where(iota == 0, fx, jnp.where(iota == 1, fy, fz))
        plsc.store_scatter(centf, [i * 3 + iota], vcent, mask=cmask)

        @plsc.parallel_loop(
            0, QN // 16, 1, unroll=4,
            carry=(jnp.full((16,), -1.0, jnp.float32),
                   jnp.full((16,), BIGI, jnp.int32)))
        def upd(j, bcarry):
            bv, bi = bcarry
            goff = qoff + j * 16
            xq = xv[pl.ds(goff, 16)]
            yq = yv[pl.ds(goff, 16)]
            zq = zv[pl.ds(goff, 16)]
            dx = xq - fx
            dy = yq - fy
            dz = zq - fz
            d2 = (dx * dx + dy * dy) + dz * dz
            dm = jnp.minimum(distv[pl.ds(j * 16, 16)], d2)
            distv[pl.ds(j * 16, 16)] = dm
            gix = goff + iota
            take = (dm > bv) | ((dm == bv) & (gix < bi))
            bv = jnp.where(take, dm, bv)
            bi = jnp.where(take, gix, bi)
            return bv, bi

        bv, bi = upd

        maxv = lax.reduce_max(bv, axes=(0,))
        cand = jnp.where(bv == maxv, bi, jnp.int32(BIGI))
        mini = lax.reduce_min(cand, axes=(0,))
        wvx = xv[pl.ds(mini, 16)]
        wvy = yv[pl.ds(mini, 16)]
        wvz = zv[pl.ds(mini, 16)]
        wx = wvx[0]
        wy = wvy[0]
        wz = wvz[0]
        pubv = jnp.where(
            iota == 0, maxv,
            jnp.where(iota == 1, mini.astype(jnp.float32),
                      jnp.where(iota == 2, wx,
                                jnp.where(iota == 3, wy, wz))))
        pub[...] = pubv
        par = i % 2
        pltpu.sync_copy(pub, shared.at[par, pl.ds(s * 16, 16)])
        plsc.subcore_barrier()
        pltpu.sync_copy(shared.at[par, pl.ds(sgrp * 16, 64)], comb)

        bvs = jnp.float32(-1.0)
        bis = jnp.float32(2.0 ** 30)
        nx = fx
        ny = fy
        nz = fz
        for w in range(4):
            row = comb[pl.ds(w * 16, 16)]
            v = row[0]
            ixf = row[1]
            better = (v > bvs) | ((v == bvs) & (ixf < bis))
            bvs = jnp.where(better, v, bvs)
            bis = jnp.where(better, ixf, bis)
            nx = jnp.where(better, row[2], nx)
            ny = jnp.where(better, row[3], ny)
            nz = jnp.where(better, row[4], nz)
        return nx, ny, nz

    v0x = xv[pl.ds(0, 16)]
    v0y = yv[pl.ds(0, 16)]
    v0z = zv[pl.ds(0, 16)]
    lax.fori_loop(0, NG, fps_step, (v0x[0], v0y[0], v0z[0]))

    boff = b * N
    r2 = jnp.float32(R2)

    def ballquery(cid, gd):
        cv = centf[pl.ds(cid * 3, 16)]
        cvb = _round_bf16(cv)
        cx = cv[0]
        cy = cv[1]
        cz = cv[2]
        ss = (cx * cx + cy * cy) + cz * cz
        cbx = cvb[0]
        cby = cvb[1]
        cbz = cvb[2]

        def chunk_mask(j):
            xq = xbv[pl.ds(j * 16, 16)]
            yq = ybv[pl.ds(j * 16, 16)]
            zq = zbv[pl.ds(j * 16, 16)]
            nq = nqv[pl.ds(j * 16, 16)]
            dot = (cbx * xq + cby * yq) + cbz * zq
            d2 = (-2.0 * dot + ss) + nq
            mask = d2 <= r2
            pc = plsc.all_reduce_population_count(mask)
            return mask, pc[0]

        def append(j, cnt, mask, pcs):
            @pl.when(pcs > 0)
            def _():
                cs = plsc.cumsum(mask.astype(jnp.int32))
                plsc.store_scatter(nbr, [(cnt - 1) + cs], j * 16 + iota,
                                   mask=mask)

        def cond(st):
            j, cnt = st
            return (cnt < K) & (j < NCHUNK)

        def scan(st):
            j, cnt = st
            m0, p0 = chunk_mask(j)
            m1, p1 = chunk_mask(j + 1)
            m2, p2 = chunk_mask(j + 2)
            m3, p3 = chunk_mask(j + 3)
            append(j, cnt, m0, p0)
            append(j + 1, cnt + p0, m1, p1)
            append(j + 2, cnt + p0 + p1, m2, p2)
            append(j + 3, cnt + p0 + p1 + p2, m3, p3)
            return j + 4, cnt + ((p0 + p1) + (p2 + p3))

        _, cnt = lax.while_loop(cond, scan, (jnp.int32(0), jnp.int32(0)))

        v1 = nbr[pl.ds(0, 16)]
        v2 = nbr[pl.ds(16, 16)]
        first = v1[0]
        gd[pl.ds(0, 16)] = jnp.where(iota < cnt, v1, first) + boff
        gd[pl.ds(16, 16)] = jnp.where(iota + 16 < cnt, v2, first) + boff

    def reduce_store(rws, si):
        accs = [rws[0, pl.ds(g * 16, 16)] for g in range(8)]
        for r in range(1, K):
            accs = [jnp.maximum(accs[g], rws[r, pl.ds(g * 16, 16)])
                    for g in range(8)]
        obase = si * C
        for g in range(8):
            outf[pl.ds(obase + g * 16, 16)] = accs[g]

    ballquery(q * SG, gidxa)
    pltpu.async_copy(f_hbm.at[gidxa], rowsa, sema)

    def pipe(si2, _):
        s0 = 2 * si2
        ballquery(q * SG + s0 + 1, gidxb)
        pltpu.async_copy(f_hbm.at[gidxb], rowsb, semb)
        pltpu.make_async_copy(f_hbm.at[gidxa], rowsa, sema).wait()
        reduce_store(rowsa, s0)

        @pl.when(s0 + 2 < SG)
        def _():
            ballquery(q * SG + s0 + 2, gidxa)
            pltpu.async_copy(f_hbm.at[gidxa], rowsa, sema)

        pltpu.make_async_copy(f_hbm.at[gidxb], rowsb, semb).wait()
        reduce_store(rowsb, s0 + 1)
        return 0

    lax.fori_loop(0, SG // 2, pipe, 0)

    pltpu.sync_copy(outf, fce_hbm.at[pl.ds((b * NG + q * SG) * C, SG * C)])

    @pl.when(q == 0)
    def _():
        pltpu.sync_copy(centf.at[pl.ds(0, NG * 3)],
                        pce_hbm.at[pl.ds(b * NG * 3, NG * 3)])


@jax.jit
def _transition_down(f, p):
    xt = jnp.transpose(p, (0, 2, 1)).reshape(B * 3 * N)
    f2 = f.reshape(B * N, C)
    mesh = plsc.VectorSubcoreMesh(
        core_axis_name="c", subcore_axis_name="s",
        num_cores=NC, num_subcores=NSUB)
    fce, pce = pl.kernel(
        _sc_body,
        out_type=(
            jax.ShapeDtypeStruct((B * NG * C,), jnp.float32),
            jax.ShapeDtypeStruct((B * NG * 3,), jnp.float32),
        ),
        mesh=mesh,
        compiler_params=pltpu.CompilerParams(needs_layout_passes=False),
        scratch_types=[
            pltpu.VMEM((N + 16,), jnp.float32),
            pltpu.VMEM((N + 16,), jnp.float32),
            pltpu.VMEM((N + 16,), jnp.float32),
            pltpu.VMEM((N,), jnp.float32),
            pltpu.VMEM((N,), jnp.float32),
            pltpu.VMEM((N,), jnp.float32),
            pltpu.VMEM((N,), jnp.float32),
            pltpu.VMEM((QN,), jnp.float32),
            pltpu.VMEM((NG * 3 + 16,), jnp.float32),
            pltpu.VMEM((16,), jnp.float32),
            pltpu.VMEM((64,), jnp.float32),
            pltpu.VMEM((112,), jnp.int32),
            pltpu.VMEM((K,), jnp.int32),
            pltpu.VMEM((K,), jnp.int32),
            pltpu.VMEM((K, C), jnp.float32),
            pltpu.VMEM((K, C), jnp.float32),
            pltpu.VMEM((SG * C,), jnp.float32),
            pltpu.VMEM_SHARED((2, NSUB * 16), jnp.float32),
            pltpu.SemaphoreType.DMA,
            pltpu.SemaphoreType.DMA,
        ],
    )(xt, f2)
    return fce.reshape(B, NG, C), pce.reshape(B, NG, 3)


def kernel(f, p):
    if f.shape[1] == NG:
        return (f, p)
    return _transition_down(f, p)


# speedup vs baseline: 1.3186x; 1.0784x over previous
import jax
import jax.numpy as jnp
from jax import lax
from jax.experimental import pallas as pl
from jax.experimental.pallas import tpu as pltpu
from jax.experimental.pallas import tpu_sc as plsc

B = 8
N = 4096
C = 128
NG = 1024
K = 32
R2 = float(0.2 ** 2)
NC = 2
NSUB = 16
QN = N // 4
NCHUNK = N // 16
SG = NG // 4
BIGI = 2 ** 31 - 1


def _iota16():
    return lax.broadcasted_iota(jnp.int32, (16,), 0)


def _round_bf16(v):
    u = plsc.bitcast(v, jnp.int32)
    r = (u + 0x7FFF + ((u >> 16) & 1)) & jnp.int32(-65536)
    return plsc.bitcast(r, jnp.float32)


def _sc_body(xt_hbm, f_hbm, fce_hbm, pce_hbm,
             xv, yv, zv, xbv, ybv, zbv, nqv, distv, centf, pub, comb, nbr,
             gidxa, gidxb, rowsa, rowsb, outf, shared, sema, semb):
    c = lax.axis_index("c")
    s = lax.axis_index("s")
    b = c * 4 + s // 4
    q = s % 4
    sgrp = (s // 4) * 4
    iota = _iota16()

    pltpu.sync_copy(xt_hbm.at[pl.ds((b * 3 + 0) * N, N)], xv.at[pl.ds(0, N)])
    pltpu.sync_copy(xt_hbm.at[pl.ds((b * 3 + 1) * N, N)], yv.at[pl.ds(0, N)])
    pltpu.sync_copy(xt_hbm.at[pl.ds((b * 3 + 2) * N, N)], zv.at[pl.ds(0, N)])

    def _init_chunk(j, _):
        xq = xv[pl.ds(j * 16, 16)]
        yq = yv[pl.ds(j * 16, 16)]
        zq = zv[pl.ds(j * 16, 16)]
        nqv[pl.ds(j * 16, 16)] = (xq * xq + yq * yq) + zq * zq
        xbv[pl.ds(j * 16, 16)] = _round_bf16(xq)
        ybv[pl.ds(j * 16, 16)] = _round_bf16(yq)
        zbv[pl.ds(j * 16, 16)] = _round_bf16(zq)
        return 0

    lax.fori_loop(0, NCHUNK, _init_chunk, 0)

    def _init_dist(j, _):
        distv[pl.ds(j * 16, 16)] = jnp.full((16,), 1e10, jnp.float32)
        return 0

    lax.fori_loop(0, QN // 16, _init_dist, 0)

    qoff = q * QN
    cmask = iota < 3

    def fps_step(i, carry):
        fx, fy, fz = carry
        vcent = jnp.where(iota == 0, fx, jnp.where(iota == 1, fy, fz))
        plsc.store_scatter(centf, [i * 3 + iota], vcent, mask=cmask)

        @plsc.parallel_loop(
            0, QN // 16, 1, unroll=4,
            carry=(jnp.full((16,), -1.0, jnp.float32),
                   jnp.full((16,), BIGI, jnp.int32)))
        def upd(j, bcarry):
            bv, bi = bcarry
            goff = qoff + j * 16
            xq = xv[pl.ds(goff, 16)]
            yq = yv[pl.ds(goff, 16)]
            zq = zv[pl.ds(goff, 16)]
            dx = xq - fx
            dy = yq - fy
            dz = zq - fz
            d2 = (dx * dx + dy * dy) + dz * dz
            dm = jnp.minimum(distv[pl.ds(j * 16, 16)], d2)
            distv[pl.ds(j * 16, 16)] = dm
            gix = goff + iota
            take = (dm > bv) | ((dm == bv) & (gix < bi))
            bv = jnp.where(take, dm, bv)
            bi = jnp.where(take, gix, bi)
            return bv, bi

        bv, bi = upd

        maxv = lax.reduce_max(bv, axes=(0,))
        cand = jnp.where(bv == maxv, bi, jnp.int32(BIGI))
        mini = lax.reduce_min(cand, axes=(0,))
        wvx = xv[pl.ds(mini, 16)]
        wvy = yv[pl.ds(mini, 16)]
        wvz = zv[pl.ds(mini, 16)]
        wx = wvx[0]
        wy = wvy[0]
        wz = wvz[0]
        pubv = jnp.where(
            iota == 0, maxv,
            jnp.where(iota == 1, mini.astype(jnp.float32),
                      jnp.where(iota == 2, wx,
                                jnp.where(iota == 3, wy, wz))))
        pub[...] = pubv
        par = i % 2
        pltpu.sync_copy(pub, shared.at[par, pl.ds(s * 16, 16)])
        plsc.subcore_barrier()
        pltpu.sync_copy(shared.at[par, pl.ds(sgrp * 16, 64)], comb)

        bvs = jnp.float32(-1.0)
        bis = jnp.float32(2.0 ** 30)
        nx = fx
        ny = fy
        nz = fz
        for w in range(4):
            row = comb[pl.ds(w * 16, 16)]
            v = row[0]
            ixf = row[1]
            better = (v > bvs) | ((v == bvs) & (ixf < bis))
            bvs = jnp.where(better, v, bvs)
            bis = jnp.where(better, ixf, bis)
            nx = jnp.where(better, row[2], nx)
            ny = jnp.where(better, row[3], ny)
            nz = jnp.where(better, row[4], nz)
        return nx, ny, nz

    v0x = xv[pl.ds(0, 16)]
    v0y = yv[pl.ds(0, 16)]
    v0z = zv[pl.ds(0, 16)]
    lax.fori_loop(0, NG, fps_step, (v0x[0], v0y[0], v0z[0]))

    boff = b * N
    r2 = jnp.float32(R2)

    def ballquery(cid, gd):
        cv = centf[pl.ds(cid * 3, 16)]
        cvb = _round_bf16(cv)
        cx = cv[0]
        cy = cv[1]
        cz = cv[2]
        ss = (cx * cx + cy * cy) + cz * cz
        cbx = cvb[0]
        cby = cvb[1]
        cbz = cvb[2]

        def chunk_mask(j):
            xq = xbv[pl.ds(j * 16, 16)]
            yq = ybv[pl.ds(j * 16, 16)]
            zq = zbv[pl.ds(j * 16, 16)]
            nq = nqv[pl.ds(j * 16, 16)]
            dot = (cbx * xq + cby * yq) + cbz * zq
            d2 = (-2.0 * dot + ss) + nq
            mask = d2 <= r2
            pc = plsc.all_reduce_population_count(mask)
            return mask, pc[0]

        def append(j, cnt, mask, pcs):
            @pl.when(pcs > 0)
            def _():
                cs = plsc.cumsum(mask.astype(jnp.int32))
                plsc.store_scatter(nbr, [(cnt - 1) + cs], j * 16 + iota,
                                   mask=mask)

        def cond(st):
            j, cnt = st
            return (cnt < K) & (j < NCHUNK)

        def scan(st):
            j, cnt = st
            mps = [chunk_mask(j + u) for u in range(8)]
            run = cnt
            for u, (mu, pu) in enumerate(mps):
                append(j + u, run, mu, pu)
                run = run + pu
            return j + 8, run

        _, cnt = lax.while_loop(cond, scan, (jnp.int32(0), jnp.int32(0)))

        v1 = nbr[pl.ds(0, 16)]
        v2 = nbr[pl.ds(16, 16)]
        first = v1[0]
        gd[pl.ds(0, 16)] = jnp.where(iota < cnt, v1, first) + boff
        gd[pl.ds(16, 16)] = jnp.where(iota + 16 < cnt, v2, first) + boff

    def reduce_store(rws, si):
        accs = [rws[0, pl.ds(g * 16, 16)] for g in range(8)]
        for r in range(1, K):
            accs = [jnp.maximum(accs[g], rws[r, pl.ds(g * 16, 16)])
                    for g in range(8)]
        obase = si * C
        for g in range(8):
            outf[pl.ds(obase + g * 16, 16)] = accs[g]

    ballquery(q * SG, gidxa)
    pltpu.async_copy(f_hbm.at[gidxa], rowsa, sema)

    def pipe(si2, _):
        s0 = 2 * si2
        ballquery(q * SG + s0 + 1, gidxb)
        pltpu.async_copy(f_hbm.at[gidxb], rowsb, semb)
        pltpu.make_async_copy(f_hbm.at[gidxa], rowsa, sema).wait()
        reduce_store(rowsa, s0)

        @pl.when(s0 + 2 < SG)
        def _():
            ballquery(q * SG + s0 + 2, gidxa)
            pltpu.async_copy(f_hbm.at[gidxa], rowsa, sema)

        pltpu.make_async_copy(f_hbm.at[gidxb], rowsb, semb).wait()
        reduce_store(rowsb, s0 + 1)
        return 0

    lax.fori_loop(0, SG // 2, pipe, 0)

    pltpu.sync_copy(outf, fce_hbm.at[pl.ds((b * NG + q * SG) * C, SG * C)])

    @pl.when(q == 0)
    def _():
        pltpu.sync_copy(centf.at[pl.ds(0, NG * 3)],
                        pce_hbm.at[pl.ds(b * NG * 3, NG * 3)])


@jax.jit
def _transition_down(f, p):
    xt = jnp.transpose(p, (0, 2, 1)).reshape(B * 3 * N)
    f2 = f.reshape(B * N, C)
    mesh = plsc.VectorSubcoreMesh(
        core_axis_name="c", subcore_axis_name="s",
        num_cores=NC, num_subcores=NSUB)
    fce, pce = pl.kernel(
        _sc_body,
        out_type=(
            jax.ShapeDtypeStruct((B * NG * C,), jnp.float32),
            jax.ShapeDtypeStruct((B * NG * 3,), jnp.float32),
        ),
        mesh=mesh,
        compiler_params=pltpu.CompilerParams(needs_layout_passes=False),
        scratch_types=[
            pltpu.VMEM((N + 16,), jnp.float32),
            pltpu.VMEM((N + 16,), jnp.float32),
            pltpu.VMEM((N + 16,), jnp.float32),
            pltpu.VMEM((N,), jnp.float32),
            pltpu.VMEM((N,), jnp.float32),
            pltpu.VMEM((N,), jnp.float32),
            pltpu.VMEM((N,), jnp.float32),
            pltpu.VMEM((QN,), jnp.float32),
            pltpu.VMEM((NG * 3 + 16,), jnp.float32),
            pltpu.VMEM((16,), jnp.float32),
            pltpu.VMEM((64,), jnp.float32),
            pltpu.VMEM((176,), jnp.int32),
            pltpu.VMEM((K,), jnp.int32),
            pltpu.VMEM((K,), jnp.int32),
            pltpu.VMEM((K, C), jnp.float32),
            pltpu.VMEM((K, C), jnp.float32),
            pltpu.VMEM((SG * C,), jnp.float32),
            pltpu.VMEM_SHARED((2, NSUB * 16), jnp.float32),
            pltpu.SemaphoreType.DMA,
            pltpu.SemaphoreType.DMA,
        ],
    )(xt, f2)
    return fce.reshape(B, NG, C), pce.reshape(B, NG, 3)


def kernel(f, p):
    if f.shape[1] == NG:
        return (f, p)
    return _transition_down(f, p)
